# baseline (speedup 1.0000x reference)
import functools

import jax
import jax.numpy as jnp
from jax import lax
from jax.experimental import pallas as pl
from jax.experimental.pallas import tpu as pltpu
from jax.experimental.pallas import tpu_sc as plsc

_VOCAB = 1_000_000
_D = 64
_B = 4096
_S = 200
_NC, _NS = 2, 16
_NW = _NC * _NS
_NBT = _B // 128
_NBLK = _S * _NBT
_PER_W = _NBLK // _NW
_TOK_W = _PER_W * 128
_VT = _VOCAB // 128
_VTAIL = _VOCAB - _VT * 128
_P1_ITERS = (_VT + _NW - 1) // _NW


@functools.partial(
    pl.kernel,
    out_type=jax.ShapeDtypeStruct((_VOCAB * _D,), jnp.float32),
    mesh=plsc.VectorSubcoreMesh(core_axis_name="c", subcore_axis_name="s"),
    compiler_params=pltpu.CompilerParams(
        use_tc_tiling_on_sc=True, needs_layout_passes=False),
    scratch_types=[
        pltpu.VMEM((8, 8, 128), jnp.float32),
        pltpu.VMEM((8, 8, 128), jnp.float32),
        pltpu.VMEM((8 * 128 * 8,), jnp.float32),
        pltpu.VMEM((8 * 128 * 8,), jnp.float32),
        pltpu.VMEM((_VTAIL * _D,), jnp.float32),
        pltpu.SemaphoreType.DMA,
        pltpu.SemaphoreType.DMA,
        pltpu.SemaphoreType.DMA,
        pltpu.SemaphoreType.DMA,
    ],
)
def _transpose_table(tabT, tail, rowmaj, in0, in1, out0, out1, tail_v,
                     isem0, isem1, osem0, osem1):
    wid = lax.axis_index("s") * _NC + lax.axis_index("c")

    @pl.when(wid == 0)
    def _():
        pltpu.sync_copy(tail, tail_v)
        pltpu.sync_copy(tail_v, rowmaj.at[pl.ds(_VT * 128 * _D, _VTAIL * _D)])

    j64 = lax.iota(jnp.int32, 16) * _D

    def fetch(v, inX, isemX):
        for g in range(8):
            pltpu.async_copy(
                tabT.at[pl.ds(8 * g, 8), pl.ds(v * 128, 128)],
                inX.at[g], isemX)

    def wait_fetch(inX, isemX):
        for g in range(8):
            pltpu.make_async_copy(
                tabT.at[pl.ds(0, 8), pl.ds(0, 128)], inX.at[g], isemX).wait()

    def trans(inX, outX):
        for g in range(8):
            for e in range(8):
                c = 8 * g + e
                for k in range(8):
                    vals = inX[g, e, pl.ds(16 * k, 16)]
                    plsc.store_scatter(outX, [j64 + (16 * k * _D + c)], vals)

    def put(v, outX, osemX):
        pltpu.async_copy(outX, rowmaj.at[pl.ds(v * 128 * _D, 128 * _D)], osemX)

    def wait_put(outX, osemX):
        pltpu.make_async_copy(
            outX, rowmaj.at[pl.ds(0, 128 * _D)], osemX).wait()

    def v_of(i):
        return wid + _NW * i

    @pl.when(v_of(0) < _VT)
    def _():
        fetch(v_of(0), in0, isem0)

    def step2(j, carry):
        i0 = 2 * j

        @pl.when(v_of(i0 + 1) < _VT)
        def _():
            fetch(v_of(i0 + 1), in1, isem1)

        @pl.when(v_of(i0) < _VT)
        def _():
            wait_fetch(in0, isem0)

            @pl.when(j >= 1)
            def _():
                wait_put(out0, osem0)

            trans(in0, out0)
            put(v_of(i0), out0, osem0)

        @pl.when(v_of(i0 + 2) < _VT)
        def _():
            fetch(v_of(i0 + 2), in0, isem0)

        @pl.when(v_of(i0 + 1) < _VT)
        def _():
            wait_fetch(in1, isem1)

            @pl.when(j >= 1)
            def _():
                wait_put(out1, osem1)

            trans(in1, out1)
            put(v_of(i0 + 1), out1, osem1)

        return carry

    lax.fori_loop(0, (_P1_ITERS + 1) // 2, step2, 0)

    @pl.when(v_of(0) < _VT)
    def _():
        wait_put(out0, osem0)

    @pl.when(v_of(1) < _VT)
    def _():
        wait_put(out1, osem1)


@functools.partial(
    pl.kernel,
    out_type=jax.ShapeDtypeStruct((_S, 8, _NBT, 8, 128), jnp.float32),
    mesh=plsc.VectorSubcoreMesh(core_axis_name="c", subcore_axis_name="s"),
    compiler_params=pltpu.CompilerParams(
        use_tc_tiling_on_sc=False, needs_layout_passes=False),
    scratch_types=[
        pltpu.VMEM((_TOK_W,), jnp.int32),
        pltpu.VMEM((128, _D), jnp.float32),
        pltpu.VMEM((128, _D), jnp.float32),
        pltpu.VMEM((8, 8, 128), jnp.float32),
        pltpu.VMEM((8, 8, 128), jnp.float32),
        pltpu.SemaphoreType.DMA,
        pltpu.SemaphoreType.DMA,
        pltpu.SemaphoreType.DMA,
        pltpu.SemaphoreType.DMA,
    ],
)
def _embed(idx_hbm, tab_hbm, out_hbm, idx_v, rows0, rows1, blk0, blk1,
           gsem0, gsem1, ssem0, ssem1):
    wid = lax.axis_index("s") * _NC + lax.axis_index("c")
    tok0 = wid * _TOK_W
    blkbase = wid * _PER_W

    pltpu.sync_copy(idx_hbm.at[pl.ds(tok0, _TOK_W)], idx_v)

    j16 = lax.iota(jnp.int32, 16)

    def gather(i, rowsX, gsemX):
        pltpu.async_copy(
            tab_hbm.at[idx_v.at[pl.ds(i * 128, 128)]], rowsX, gsemX)

    def wait_gather(rowsX, gsemX):
        pltpu.make_async_copy(tab_hbm.at[pl.ds(0, 128)], rowsX, gsemX).wait()

    def put(i, blkX, ssemX):
        k = blkbase + i
        s = k // _NBT
        bt = lax.rem(k, _NBT)
        pltpu.async_copy(blkX, out_hbm.at[s, :, bt], ssemX)

    def wait_put(blkX, ssemX):
        pltpu.make_async_copy(blkX, out_hbm.at[0, :, 0], ssemX).wait()

    def transpose(rowsX, blkX):
        for g in range(8):
            for e in range(8):
                cvec = jnp.full((16,), 8 * g + e, jnp.int32)
                for k in range(8):
                    vals = plsc.load_gather(rowsX, [j16 + 16 * k, cvec])
                    blkX[g, e, pl.ds(16 * k, 16)] = vals

    gather(0, rows0, gsem0)

    def step2(j, carry):
        i0 = 2 * j

        @pl.when(i0 + 1 < _PER_W)
        def _():
            gather(i0 + 1, rows1, gsem1)

        wait_gather(rows0, gsem0)

        @pl.when(j >= 1)
        def _():
            wait_put(blk0, ssem0)

        transpose(rows0, blk0)
        put(i0, blk0, ssem0)

        @pl.when(i0 + 2 < _PER_W)
        def _():
            gather(i0 + 2, rows0, gsem0)

        wait_gather(rows1, gsem1)

        @pl.when(j >= 1)
        def _():
            wait_put(blk1, ssem1)

        transpose(rows1, blk1)
        put(i0 + 1, blk1, ssem1)
        return carry

    lax.fori_loop(0, _PER_W // 2, step2, 0)
    wait_put(blk0, ssem0)
    wait_put(blk1, ssem1)


def kernel(tokens, table):
    idx = tokens.T.reshape(-1).astype(jnp.int32)
    tab_t = table.T
    tail = lax.slice(table, (_VT * 128, 0), (_VOCAB, _D)).reshape(-1)
    rowmaj = _transpose_table(tab_t, tail)
    out5 = _embed(idx, rowmaj.reshape(_VOCAB, _D))
    return out5.transpose(2, 4, 0, 1, 3).reshape(_B, _S, _D)


# speedup vs baseline: 1.1974x; 1.1974x over previous
import functools

import jax
import jax.numpy as jnp
from jax import lax
from jax.experimental import pallas as pl
from jax.experimental.pallas import tpu as pltpu
from jax.experimental.pallas import tpu_sc as plsc

_VOCAB = 1_000_000
_D = 64
_B = 4096
_S = 200
_NC, _NS = 2, 16
_NW = _NC * _NS
_NBT = _B // 128
_NBLK = _S * _NBT
_PER_W = _NBLK // _NW
_TOK_W = _PER_W * 128
_ND = _PER_W // 2
_VT = _VOCAB // 128
_VTAIL = _VOCAB - _VT * 128
_P1_ITERS = (_VT + _NW - 1) // _NW


@functools.partial(
    pl.kernel,
    out_type=jax.ShapeDtypeStruct((_VOCAB * _D,), jnp.float32),
    mesh=plsc.VectorSubcoreMesh(core_axis_name="c", subcore_axis_name="s"),
    compiler_params=pltpu.CompilerParams(
        use_tc_tiling_on_sc=True, needs_layout_passes=False),
    scratch_types=[
        pltpu.VMEM((8, 8, 128), jnp.float32),
        pltpu.VMEM((8, 8, 128), jnp.float32),
        pltpu.VMEM((8 * 128 * 8,), jnp.float32),
        pltpu.VMEM((8 * 128 * 8,), jnp.float32),
        pltpu.VMEM((_VTAIL * _D,), jnp.float32),
        pltpu.SemaphoreType.DMA,
        pltpu.SemaphoreType.DMA,
        pltpu.SemaphoreType.DMA,
        pltpu.SemaphoreType.DMA,
    ],
)
def _transpose_table(tabT, tail, rowmaj, in0, in1, out0, out1, tail_v,
                     isem0, isem1, osem0, osem1):
    wid = lax.axis_index("s") * _NC + lax.axis_index("c")

    @pl.when(wid == 0)
    def _():
        pltpu.sync_copy(tail, tail_v)
        pltpu.sync_copy(tail_v, rowmaj.at[pl.ds(_VT * 128 * _D, _VTAIL * _D)])

    j64 = lax.iota(jnp.int32, 16) * _D

    def fetch(v, inX, isemX):
        for g in range(8):
            pltpu.async_copy(
                tabT.at[pl.ds(8 * g, 8), pl.ds(v * 128, 128)],
                inX.at[g], isemX)

    def wait_fetch(inX, isemX):
        for g in range(8):
            pltpu.make_async_copy(
                tabT.at[pl.ds(0, 8), pl.ds(0, 128)], inX.at[g], isemX).wait()

    def trans(inX, outX):
        for g in range(8):
            for e in range(8):
                c = 8 * g + e
                vals = [inX[g, e, pl.ds(16 * k, 16)] for k in range(8)]
                for k in range(8):
                    plsc.store_scatter(
                        outX, [j64 + (16 * k * _D + c)], vals[k])

    def put(v, outX, osemX):
        pltpu.async_copy(outX, rowmaj.at[pl.ds(v * 128 * _D, 128 * _D)], osemX)

    def wait_put(outX, osemX):
        pltpu.make_async_copy(
            outX, rowmaj.at[pl.ds(0, 128 * _D)], osemX).wait()

    def v_of(i):
        return wid + _NW * i

    @pl.when(v_of(0) < _VT)
    def _():
        fetch(v_of(0), in0, isem0)

    def step2(j, carry):
        i0 = 2 * j

        @pl.when(v_of(i0 + 1) < _VT)
        def _():
            fetch(v_of(i0 + 1), in1, isem1)

        @pl.when(v_of(i0) < _VT)
        def _():
            wait_fetch(in0, isem0)

            @pl.when(j >= 1)
            def _():
                wait_put(out0, osem0)

            trans(in0, out0)
            put(v_of(i0), out0, osem0)

        @pl.when(v_of(i0 + 2) < _VT)
        def _():
            fetch(v_of(i0 + 2), in0, isem0)

        @pl.when(v_of(i0 + 1) < _VT)
        def _():
            wait_fetch(in1, isem1)

            @pl.when(j >= 1)
            def _():
                wait_put(out1, osem1)

            trans(in1, out1)
            put(v_of(i0 + 1), out1, osem1)

        return carry

    lax.fori_loop(0, (_P1_ITERS + 1) // 2, step2, 0)

    @pl.when(v_of(0) < _VT)
    def _():
        wait_put(out0, osem0)

    @pl.when(v_of(1) < _VT)
    def _():
        wait_put(out1, osem1)


@functools.partial(
    pl.kernel,
    out_type=jax.ShapeDtypeStruct((_S, 8, _NBT, 8, 128), jnp.float32),
    mesh=plsc.VectorSubcoreMesh(core_axis_name="c", subcore_axis_name="s"),
    compiler_params=pltpu.CompilerParams(
        use_tc_tiling_on_sc=False, needs_layout_passes=False),
    scratch_types=[
        pltpu.VMEM((_TOK_W,), jnp.int32),
        pltpu.VMEM((256, _D), jnp.float32),
        pltpu.VMEM((256, _D), jnp.float32),
        pltpu.VMEM((8, 8, 128), jnp.float32),
        pltpu.VMEM((8, 8, 128), jnp.float32),
        pltpu.VMEM((8, 8, 128), jnp.float32),
        pltpu.VMEM((8, 8, 128), jnp.float32),
        pltpu.SemaphoreType.DMA,
        pltpu.SemaphoreType.DMA,
        pltpu.SemaphoreType.DMA,
        pltpu.SemaphoreType.DMA,
        pltpu.SemaphoreType.DMA,
        pltpu.SemaphoreType.DMA,
    ],
)
def _embed(idx_hbm, tab_hbm, out_hbm, idx_v, rowsA, rowsB,
           blk0, blk1, blk2, blk3, gsemA, gsemB, sem0, sem1, sem2, sem3):
    wid = lax.axis_index("s") * _NC + lax.axis_index("c")
    tok0 = wid * _TOK_W
    blkbase = wid * _PER_W

    pltpu.sync_copy(idx_hbm.at[pl.ds(tok0, _TOK_W)], idx_v)

    j16 = lax.iota(jnp.int32, 16)

    def gather(d, rowsX, gsemX):
        pltpu.async_copy(
            tab_hbm.at[idx_v.at[pl.ds(d * 256, 256)]], rowsX, gsemX)

    def wait_gather(rowsX, gsemX):
        pltpu.make_async_copy(tab_hbm.at[pl.ds(0, 256)], rowsX, gsemX).wait()

    def put(i, blkX, semX):
        k = blkbase + i
        s = k // _NBT
        bt = lax.rem(k, _NBT)
        pltpu.async_copy(blkX, out_hbm.at[s, :, bt], semX)

    def wait_put(blkX, semX):
        pltpu.make_async_copy(blkX, out_hbm.at[0, :, 0], semX).wait()

    def transpose(rowsX, half, blkX):
        for g in range(8):
            for e in range(8):
                cvec = jnp.full((16,), 8 * g + e, jnp.int32)
                vals = [
                    plsc.load_gather(
                        rowsX, [j16 + (128 * half + 16 * k), cvec])
                    for k in range(8)
                ]
                for k in range(8):
                    blkX[g, e, pl.ds(16 * k, 16)] = vals[k]

    gather(0, rowsA, gsemA)

    def step2(u, carry):
        d0 = 2 * u
        d1 = d0 + 1

        gather(d1, rowsB, gsemB)
        wait_gather(rowsA, gsemA)

        @pl.when(u >= 1)
        def _():
            wait_put(blk0, sem0)
            wait_put(blk1, sem1)

        transpose(rowsA, 0, blk0)
        put(2 * d0, blk0, sem0)
        transpose(rowsA, 1, blk1)
        put(2 * d0 + 1, blk1, sem1)

        @pl.when(d0 + 2 < _ND)
        def _():
            gather(d0 + 2, rowsA, gsemA)

        wait_gather(rowsB, gsemB)

        @pl.when(u >= 1)
        def _():
            wait_put(blk2, sem2)
            wait_put(blk3, sem3)

        transpose(rowsB, 0, blk2)
        put(2 * d1, blk2, sem2)
        transpose(rowsB, 1, blk3)
        put(2 * d1 + 1, blk3, sem3)
        return carry

    lax.fori_loop(0, _ND // 2, step2, 0)
    wait_put(blk0, sem0)
    wait_put(blk1, sem1)
    wait_put(blk2, sem2)
    wait_put(blk3, sem3)


def kernel(tokens, table):
    idx = tokens.T.reshape(-1).astype(jnp.int32)
    tab_t = table.T
    tail = lax.slice(table, (_VT * 128, 0), (_VOCAB, _D)).reshape(-1)
    rowmaj = _transpose_table(tab_t, tail)
    out5 = _embed(idx, rowmaj.reshape(_VOCAB, _D))
    return out5.transpose(2, 4, 0, 1, 3).reshape(_B, _S, _D)


# speedup vs baseline: 2.4036x; 2.0074x over previous
import functools

import jax
import jax.numpy as jnp
from jax import lax
from jax.experimental import pallas as pl
from jax.experimental.pallas import tpu as pltpu
from jax.experimental.pallas import tpu_sc as plsc

_VOCAB = 1_000_000
_D = 64
_B = 4096
_S = 200
_NC, _NS = 2, 16
_NW = _NC * _NS
_NBT = _B // 128
_NBLK = _S * _NBT
_PER_W = _NBLK // _NW
_TOK_W = _PER_W * 128
_ND = _PER_W // 2
_VT = _VOCAB // 128
_VTAIL = _VOCAB - _VT * 128
_P1_ITERS = (_VT + _NW - 1) // _NW


@functools.partial(
    pl.kernel,
    out_type=jax.ShapeDtypeStruct((_VOCAB * _D,), jnp.float32),
    mesh=plsc.VectorSubcoreMesh(core_axis_name="c", subcore_axis_name="s"),
    compiler_params=pltpu.CompilerParams(
        use_tc_tiling_on_sc=True, needs_layout_passes=False),
    scratch_types=[
        pltpu.VMEM((_D, 128), jnp.float32),
        pltpu.VMEM((_D, 128), jnp.float32),
        pltpu.VMEM((128 * 65,), jnp.float32),
        pltpu.VMEM((8 * 128 * 8,), jnp.float32),
        pltpu.VMEM((8 * 128 * 8,), jnp.float32),
        pltpu.VMEM((_VTAIL * _D,), jnp.float32),
        pltpu.SemaphoreType.DMA,
        pltpu.SemaphoreType.DMA,
        pltpu.SemaphoreType.DMA,
        pltpu.SemaphoreType.DMA,
    ],
)
def _transpose_table(tabT, tail, rowmaj, in0, in1, mid, out0, out1, tail_v,
                     isem0, isem1, osem0, osem1):
    wid = lax.axis_index("s") * _NC + lax.axis_index("c")

    @pl.when(wid == 0)
    def _():
        pltpu.sync_copy(tail, tail_v)
        pltpu.sync_copy(tail_v, rowmaj.at[pl.ds(_VT * 128 * _D, _VTAIL * _D)])

    j16 = lax.iota(jnp.int32, 16)
    j65 = [(j16 + 16 * k) * 65 for k in range(8)]

    def fetch(v, inX, isemX):
        pltpu.async_copy(
            tabT.at[pl.ds(0, _D), pl.ds(v * 128, 128)], inX, isemX)

    def wait_fetch(inX, isemX):
        pltpu.make_async_copy(
            tabT.at[pl.ds(0, _D), pl.ds(0, 128)], inX, isemX).wait()

    def trans(inX, outX):
        for c in range(_D):
            vals = [inX[c, pl.ds(16 * k, 16)] for k in range(8)]
            for k in range(8):
                plsc.store_scatter(mid, [j65[k] + c], vals[k])
        for j in range(128):
            vals = [mid[pl.ds(j * 65 + 16 * m, 16)] for m in range(4)]
            for m in range(4):
                outX[pl.ds(j * _D + 16 * m, 16)] = vals[m]

    def put(v, outX, osemX):
        pltpu.async_copy(outX, rowmaj.at[pl.ds(v * 128 * _D, 128 * _D)], osemX)

    def wait_put(outX, osemX):
        pltpu.make_async_copy(
            outX, rowmaj.at[pl.ds(0, 128 * _D)], osemX).wait()

    def v_of(i):
        return wid + _NW * i

    @pl.when(v_of(0) < _VT)
    def _():
        fetch(v_of(0), in0, isem0)

    def step2(j, carry):
        i0 = 2 * j

        @pl.when(v_of(i0 + 1) < _VT)
        def _():
            fetch(v_of(i0 + 1), in1, isem1)

        @pl.when(v_of(i0) < _VT)
        def _():
            wait_fetch(in0, isem0)

            @pl.when(j >= 1)
            def _():
                wait_put(out0, osem0)

            trans(in0, out0)
            put(v_of(i0), out0, osem0)

        @pl.when(v_of(i0 + 2) < _VT)
        def _():
            fetch(v_of(i0 + 2), in0, isem0)

        @pl.when(v_of(i0 + 1) < _VT)
        def _():
            wait_fetch(in1, isem1)

            @pl.when(j >= 1)
            def _():
                wait_put(out1, osem1)

            trans(in1, out1)
            put(v_of(i0 + 1), out1, osem1)

        return carry

    lax.fori_loop(0, (_P1_ITERS + 1) // 2, step2, 0)

    @pl.when(v_of(0) < _VT)
    def _():
        wait_put(out0, osem0)

    @pl.when(v_of(1) < _VT)
    def _():
        wait_put(out1, osem1)


@functools.partial(
    pl.kernel,
    out_type=jax.ShapeDtypeStruct((_S, 8, _NBT, 8, 128), jnp.float32),
    mesh=plsc.VectorSubcoreMesh(core_axis_name="c", subcore_axis_name="s"),
    compiler_params=pltpu.CompilerParams(
        use_tc_tiling_on_sc=False, needs_layout_passes=False),
    scratch_types=[
        pltpu.VMEM((_TOK_W,), jnp.int32),
        pltpu.VMEM((256, _D), jnp.float32),
        pltpu.VMEM((256, _D), jnp.float32),
        pltpu.VMEM((8, 8, 129), jnp.float32),
        pltpu.VMEM((8, 8, 129), jnp.float32),
        pltpu.VMEM((8, 8, 129), jnp.float32),
        pltpu.VMEM((8, 8, 129), jnp.float32),
        pltpu.SemaphoreType.DMA,
        pltpu.SemaphoreType.DMA,
        pltpu.SemaphoreType.DMA,
        pltpu.SemaphoreType.DMA,
        pltpu.SemaphoreType.DMA,
        pltpu.SemaphoreType.DMA,
    ],
)
def _embed(idx_hbm, tab_hbm, out_hbm, idx_v, rowsA, rowsB,
           blk0, blk1, blk2, blk3, gsemA, gsemB, sem0, sem1, sem2, sem3):
    wid = lax.axis_index("s") * _NC + lax.axis_index("c")
    tok0 = wid * _TOK_W
    blkbase = wid * _PER_W

    pltpu.sync_copy(idx_hbm.at[pl.ds(tok0, _TOK_W)], idx_v)

    i129 = lax.iota(jnp.int32, 16) * 129

    def gather(d, rowsX, gsemX):
        pltpu.async_copy(
            tab_hbm.at[idx_v.at[pl.ds(d * 256, 256)]], rowsX, gsemX)

    def wait_gather(rowsX, gsemX):
        pltpu.make_async_copy(tab_hbm.at[pl.ds(0, 256)], rowsX, gsemX).wait()

    def put(i, blkX, semX):
        k = blkbase + i
        s = k // _NBT
        bt = lax.rem(k, _NBT)
        pltpu.async_copy(
            blkX.at[:, :, pl.ds(0, 128)], out_hbm.at[s, :, bt], semX)

    def wait_put(blkX, semX):
        pltpu.make_async_copy(
            blkX.at[:, :, pl.ds(0, 128)], out_hbm.at[0, :, 0], semX).wait()

    c16 = lax.iota(jnp.int32, 16)
    gvs = [(c16 + 16 * m) >> 3 for m in range(4)]
    evs = [(c16 + 16 * m) & 7 for m in range(4)]

    def transpose(rowsX, half, blkX):
        for j in range(128):
            jv = jnp.full((16,), j, jnp.int32)
            vals = [rowsX[128 * half + j, pl.ds(16 * m, 16)]
                    for m in range(4)]
            for m in range(4):
                plsc.store_scatter(blkX, [gvs[m], evs[m], jv], vals[m])

    gather(0, rowsA, gsemA)

    def step2(u, carry):
        d0 = 2 * u
        d1 = d0 + 1

        gather(d1, rowsB, gsemB)
        wait_gather(rowsA, gsemA)

        @pl.when(u >= 1)
        def _():
            wait_put(blk0, sem0)
            wait_put(blk1, sem1)

        transpose(rowsA, 0, blk0)
        put(2 * d0, blk0, sem0)
        transpose(rowsA, 1, blk1)
        put(2 * d0 + 1, blk1, sem1)

        @pl.when(d0 + 2 < _ND)
        def _():
            gather(d0 + 2, rowsA, gsemA)

        wait_gather(rowsB, gsemB)

        @pl.when(u >= 1)
        def _():
            wait_put(blk2, sem2)
            wait_put(blk3, sem3)

        transpose(rowsB, 0, blk2)
        put(2 * d1, blk2, sem2)
        transpose(rowsB, 1, blk3)
        put(2 * d1 + 1, blk3, sem3)
        return carry

    lax.fori_loop(0, _ND // 2, step2, 0)
    wait_put(blk0, sem0)
    wait_put(blk1, sem1)
    wait_put(blk2, sem2)
    wait_put(blk3, sem3)


def kernel(tokens, table):
    idx = tokens.T.reshape(-1).astype(jnp.int32)
    tab_t = table.T
    tail = lax.slice(table, (_VT * 128, 0), (_VOCAB, _D)).reshape(-1)
    rowmaj = _transpose_table(tab_t, tail)
    out5 = _embed(idx, rowmaj.reshape(_VOCAB, _D))
    return out5.transpose(2, 4, 0, 1, 3).reshape(_B, _S, _D)
